# baseline (speedup 1.0000x reference)
import functools

import jax
import jax.numpy as jnp
from jax import lax
from jax.experimental import pallas as pl
from jax.experimental.pallas import tpu as pltpu
from jax.experimental.pallas import tpu_sc as plsc

_NC = 2
_NS = 16
_LN = 16
_EC = 128
_NCH = 8
_SUP = 8


def _cdiv(a, b):
    return (a + b - 1) // b


@functools.partial(jax.jit, static_argnames=("B", "N", "D", "E", "K"))
def _sc_aggregate(X2, sel, src, dst, B, N, D, E, K):
    assert B % _NC == 0 and N % _NCH == 0 and E % _EC == 0
    assert K == _LN and D % _LN == 0
    bpc = B // _NC
    n_chunks = N // _NCH
    e_chunks = E // _EC
    cpt_n = _cdiv(n_chunks, _NS)
    cpt_e = _cdiv(e_chunks, _NS)
    dcols = D // _LN
    nza = N // _NS
    nzd = _cdiv(cpt_n * _NCH, _LN) * _LN

    sup_e = _cdiv(cpt_e, _SUP) + 1
    sup_n = _cdiv(cpt_n, _SUP) + 1
    src_p = jnp.pad(src, (0, _NS * sup_e * _SUP * _EC - E))
    dst_p = jnp.pad(dst, (0, _NS * sup_e * _SUP * _EC - E))
    sel_p = jnp.pad(sel, ((0, _NS * sup_n * _SUP * _NCH - N), (0, 0)))

    mesh = plsc.VectorSubcoreMesh(core_axis_name="c", subcore_axis_name="s")

    def body(x_ref, sel_ref, src_ref, dst_ref, out_ref, acce_ref, dege_ref,
             ssrc, sdst, ssel, gidx0, gidx1, dstb0, dstb1, idxb0, idxb1,
             onesbuf, rows0, rows1, nodebuf0, nodebuf1, zvec, acc_sh, deg_sh,
             sem0, sem1, wsem0, wsem1):
        c = lax.axis_index("c")
        t = lax.axis_index("s")
        gidxs = (gidx0, gidx1)
        dstbs = (dstb0, dstb1)
        idxbs = (idxb0, idxb1)
        rowss = (rows0, rows1)
        sems = (sem0, sem1)
        nodebufs = (nodebuf0, nodebuf1)
        wsems = (wsem0, wsem1)

        def fill_ones(i, carry):
            onesbuf[pl.ds(i * _LN, _LN)] = jnp.ones((_LN,), jnp.float32)
            return carry
        lax.fori_loop(0, _EC // _LN, fill_ones, 0)

        def fill_zvec(i, carry):
            zvec[pl.ds(i * _LN, _LN)] = jnp.zeros((_LN,), jnp.float32)
            return carry
        lax.fori_loop(0, nzd // _LN, fill_zvec, 0)

        for bi in range(bpc):
            b = c * bpc + bi
            bN = b * N
            bN2 = b * (_NS * nzd)

            def zero_rows(i, carry):
                def zr(dc, cr):
                    rows0[i, pl.ds(dc * _LN, _LN)] = jnp.zeros(
                        (_LN,), jnp.float32)
                    return cr
                lax.fori_loop(0, dcols, zr, 0)
                return carry
            lax.fori_loop(0, _EC, zero_rows, 0)
            nfull = nza // _EC
            for j in range(nfull):
                pltpu.sync_copy(
                    rows0, acc_sh.at[pl.ds(t * nza + j * _EC, _EC)])
            rem = nza - nfull * _EC
            if rem:
                pltpu.sync_copy(
                    rows0.at[pl.ds(0, rem)],
                    acc_sh.at[pl.ds(t * nza + nfull * _EC, rem)])
            pltpu.sync_copy(zvec, deg_sh.at[pl.ds(t * nzd, nzd)])
            plsc.subcore_barrier()

            def e_super(s):
                off = (t * cpt_e + s * _SUP) * _EC
                pltpu.sync_copy(src_ref.at[pl.ds(off, _SUP * _EC)], ssrc)
                pltpu.sync_copy(dst_ref.at[pl.ds(off, _SUP * _EC)], sdst)

            def e_ok(i):
                return (i < cpt_e) & (t * cpt_e + i < e_chunks)

            def e_issue(i, p):
                @pl.when(e_ok(i))
                def _():
                    koff = (i % _SUP) * _EC

                    def adj(r, cr):
                        sl = pl.ds(r * _LN, _LN)
                        gidxs[p][sl] = ssrc[pl.ds(koff + r * _LN, _LN)] + bN
                        dstbs[p][sl] = sdst[pl.ds(koff + r * _LN, _LN)]
                        return cr
                    lax.fori_loop(0, _EC // _LN, adj, 0)
                    pltpu.async_copy(x_ref.at[gidxs[p]], rowss[p], sems[p])

            def e_drain(i, p):
                @pl.when(e_ok(i))
                def _():
                    pltpu.make_async_copy(
                        x_ref.at[gidxs[p]], rowss[p], sems[p]).wait()
                    pltpu.sync_copy(rowss[p], acc_sh.at[dstbs[p]], add=True)
                    pltpu.sync_copy(onesbuf, deg_sh.at[dstbs[p]], add=True)

            e_super(0)
            e_issue(0, 0)

            def edge_pair(i2, carry):
                i0 = 2 * i2
                e_issue(i0 + 1, 1)
                e_drain(i0, 0)

                @pl.when((i0 + 2) % _SUP == 0)
                def _():
                    e_super((i0 + 2) // _SUP)
                e_issue(i0 + 2, 0)
                e_drain(i0 + 1, 1)
                return carry
            lax.fori_loop(0, _cdiv(cpt_e, 2), edge_pair, 0)

            def a_super(s):
                off = (t * cpt_n + s * _SUP) * _NCH
                pltpu.sync_copy(sel_ref.at[pl.ds(off, _SUP * _NCH)], ssel)

            def a_ok(i):
                return ((i >= 0) & (i < cpt_n)
                        & (t * cpt_n + i < n_chunks))

            def a_issue(i, p):
                @pl.when(a_ok(i))
                def _():
                    roff = (i % _SUP) * _NCH

                    def adj(r, cr):
                        idxbs[p][pl.ds(r * K, K)] = ssel[roff + r, :] + bN
                        return cr
                    lax.fori_loop(0, _NCH, adj, 0)
                    pltpu.async_copy(x_ref.at[idxbs[p]], rowss[p], sems[p])

            def a_wwait(i, p):
                @pl.when(a_ok(i))
                def _():
                    node0 = (t * cpt_n + i) * _NCH
                    pltpu.make_async_copy(
                        nodebufs[p],
                        out_ref.at[pl.ds(bN + node0, _NCH)],
                        wsems[p]).wait()

            def a_drain(i, p):
                a_wwait(i - 2, p)

                @pl.when(a_ok(i))
                def _():
                    pltpu.make_async_copy(
                        x_ref.at[idxbs[p]], rowss[p], sems[p]).wait()
                    ch = t * cpt_n + i
                    node0 = ch * _NCH
                    rows = rowss[p]

                    def red(r, cr):
                        base = r * K
                        for dc in range(dcols):
                            sl = pl.ds(dc * _LN, _LN)
                            s = rows[base, sl]
                            for k in range(1, K):
                                s = s + rows[base + k, sl]
                            nodebufs[p][r, sl] = s * (1.0 / K)
                        return cr
                    lax.fori_loop(0, _NCH, red, 0)
                    pltpu.async_copy(
                        nodebufs[p], out_ref.at[pl.ds(bN + node0, _NCH)],
                        wsems[p])

            a_super(0)
            a_issue(0, 0)

            def node_pair(i2, carry):
                i0 = 2 * i2
                a_issue(i0 + 1, 1)
                a_drain(i0, 0)

                @pl.when((i0 + 2) % _SUP == 0)
                def _():
                    a_super((i0 + 2) // _SUP)
                a_issue(i0 + 2, 0)
                a_drain(i0 + 1, 1)
                return carry
            lax.fori_loop(0, _cdiv(cpt_n, 2), node_pair, 0)
            last = 2 * _cdiv(cpt_n, 2)
            a_wwait(last - 2, 0)
            a_wwait(last - 1, 1)
            plsc.subcore_barrier()

            cpt_x = _cdiv(_cdiv(N, _EC), _NS)
            for j in range(cpt_x):
                xch = t * cpt_x + j

                @pl.when(xch * _EC + _EC <= N)
                def _():
                    o = xch * _EC
                    pltpu.sync_copy(acc_sh.at[pl.ds(o, _EC)],
                                    acce_ref.at[pl.ds(bN + o, _EC)])
            rem_off = (N // _EC) * _EC
            rem_len = N - rem_off
            if rem_len:
                @pl.when(t == _NS - 1)
                def _():
                    pltpu.sync_copy(
                        acc_sh.at[pl.ds(rem_off, rem_len)],
                        acce_ref.at[pl.ds(bN + rem_off, rem_len)])
            pltpu.sync_copy(deg_sh.at[pl.ds(t * nzd, nzd)],
                            dege_ref.at[pl.ds(bN2 + t * nzd, nzd)])
            if bi + 1 < bpc:
                plsc.subcore_barrier()

    ndeg = _NS * nzd
    fn = pl.kernel(
        body,
        out_type=(jax.ShapeDtypeStruct((B * N, D), jnp.float32),
                  jax.ShapeDtypeStruct((B * N, D), jnp.float32),
                  jax.ShapeDtypeStruct((B * ndeg,), jnp.float32)),
        mesh=mesh,
        scratch_types=[
            pltpu.VMEM((_SUP * _EC,), jnp.int32),
            pltpu.VMEM((_SUP * _EC,), jnp.int32),
            pltpu.VMEM((_SUP * _NCH, K), jnp.int32),
            pltpu.VMEM((_EC,), jnp.int32),
            pltpu.VMEM((_EC,), jnp.int32),
            pltpu.VMEM((_EC,), jnp.int32),
            pltpu.VMEM((_EC,), jnp.int32),
            pltpu.VMEM((_NCH * K,), jnp.int32),
            pltpu.VMEM((_NCH * K,), jnp.int32),
            pltpu.VMEM((_EC,), jnp.float32),
            pltpu.VMEM((_EC, D), jnp.float32),
            pltpu.VMEM((_EC, D), jnp.float32),
            pltpu.VMEM((_NCH, D), jnp.float32),
            pltpu.VMEM((_NCH, D), jnp.float32),
            pltpu.VMEM((nzd,), jnp.float32),
            pltpu.VMEM_SHARED((N, D), jnp.float32),
            pltpu.VMEM_SHARED((_NS * nzd,), jnp.float32),
            pltpu.SemaphoreType.DMA,
            pltpu.SemaphoreType.DMA,
            pltpu.SemaphoreType.DMA,
            pltpu.SemaphoreType.DMA,
        ],
    )
    mean2, acce2, dege = fn(X2, sel_p, src_p, dst_p)
    deg2 = dege.reshape(B, ndeg)[:, :N].reshape(B * N, 1)
    return mean2, acce2, deg2


def _tc_body(mean_ref, acc_ref, deg_ref, h_ref, wz, bz, lwzt, lwzb, lbz,
             wr, br, lwrt, lwrb, lbr, wh, bh, lwht, lwhb, lbh, out_ref):
    a = mean_ref[...] + acc_ref[...] / jnp.maximum(deg_ref[...], 1.0)
    h = h_ref[...]
    dot = functools.partial(jnp.dot, preferred_element_type=jnp.float32)
    hz = jnp.maximum(dot(a, wz[...]) + bz[...], 0.0)
    hr = jnp.maximum(dot(a, wr[...]) + br[...], 0.0)
    hh = jnp.maximum(dot(a, wh[...]) + bh[...], 0.0)
    z = jax.nn.sigmoid(dot(hz, lwzt[...]) + dot(h, lwzb[...]) + lbz[...])
    r = jax.nn.sigmoid(dot(hr, lwrt[...]) + dot(h, lwrb[...]) + lbr[...])
    ht = jnp.tanh(dot(hh, lwht[...]) + dot(h * r, lwhb[...]) + lbh[...])
    out_ref[...] = z * h + (1.0 - z) * ht


def _tc_dense(mean2, acc2, deg2, H2,
              Wz, bz, LWz, Lbz, Wr, br, LWr, Lbr, Wh, bh, LWh, Lbh):
    M, DO = H2.shape
    DI = mean2.shape[1]
    rblk = 2000
    assert M % rblk == 0
    grid = (M // rblk,)
    row = pl.BlockSpec((rblk, DI), lambda i: (i, 0))
    rowd = pl.BlockSpec((rblk, 1), lambda i: (i, 0))
    roww = pl.BlockSpec((rblk, DO), lambda i: (i, 0))
    wfull = lambda s: pl.BlockSpec(s, lambda i: (0, 0))
    b2 = lambda v: v.reshape(1, DO)
    return pl.pallas_call(
        _tc_body,
        grid=grid,
        in_specs=[row, row, rowd, roww] + [wfull((DI, DO)), wfull((1, DO)),
                                wfull((DO, DO)), wfull((DO, DO)), wfull((1, DO))] * 3,
        out_specs=roww,
        out_shape=jax.ShapeDtypeStruct((M, DO), jnp.float32),
    )(mean2, acc2, deg2, H2,
      Wz, b2(bz), LWz[:DO], LWz[DO:], b2(Lbz),
      Wr, b2(br), LWr[:DO], LWr[DO:], b2(Lbr),
      Wh, b2(bh), LWh[:DO], LWh[DO:], b2(Lbh))


def kernel(X, H, Wz, bz, LWz, Lbz, Wr, br, LWr, Lbr, Wh, bh, LWh, Lbh,
           edge_index, selections):
    B, N, DI = X.shape
    DO = H.shape[-1]
    E = edge_index.shape[1]
    K = selections.shape[1]
    X2 = X.reshape(B * N, DI)
    mean2, acc2, deg2 = _sc_aggregate(X2, selections, edge_index[0],
                                      edge_index[1], B=B, N=N, D=DI, E=E, K=K)
    out2 = _tc_dense(mean2, acc2, deg2, H.reshape(B * N, DO),
                     Wz, bz, LWz, Lbz, Wr, br, LWr, Lbr, Wh, bh, LWh, Lbh)
    return out2.reshape(B, N, DO)


# speedup vs baseline: 1.0363x; 1.0363x over previous
import functools

import jax
import jax.numpy as jnp
from jax import lax
from jax.experimental import pallas as pl
from jax.experimental.pallas import tpu as pltpu
from jax.experimental.pallas import tpu_sc as plsc

_NC = 2
_NS = 16
_LN = 16
_EC = 128
_NCH = 8
_SUP = 8


def _cdiv(a, b):
    return (a + b - 1) // b


@functools.partial(jax.jit, static_argnames=("B", "N", "D", "E", "K"))
def _sc_aggregate(X2, sel, src, dst, B, N, D, E, K):
    assert B % _NC == 0 and N % _NCH == 0 and E % _EC == 0
    assert K == _LN and D % _LN == 0
    bpc = B // _NC
    n_chunks = N // _NCH
    e_chunks = E // _EC
    cpt_n = _cdiv(n_chunks, _NS)
    cpt_e = _cdiv(e_chunks, _NS)
    dcols = D // _LN
    nza = N // _NS
    nzd = _cdiv(cpt_n * _NCH, _LN) * _LN

    sup_e = _cdiv(cpt_e, _SUP) + 1
    sup_n = _cdiv(cpt_n, _SUP) + 1
    src_p = jnp.pad(src, (0, _NS * sup_e * _SUP * _EC - E))
    dst_p = jnp.pad(dst, (0, _NS * sup_e * _SUP * _EC - E))
    sel_p = jnp.pad(sel, ((0, _NS * sup_n * _SUP * _NCH - N), (0, 0)))

    mesh = plsc.VectorSubcoreMesh(core_axis_name="c", subcore_axis_name="s")

    def body(x_ref, sel_ref, src_ref, dst_ref, out_ref,
             ssrc, sdst, ssel, gidx0, gidx1, dstb0, dstb1, idxb0, idxb1,
             onesbuf, rows0, rows1, nodebuf0, nodebuf1, accbuf0, accbuf1,
             degv0, degv1, invbuf, zvec, acc_sh, deg_sh,
             sem0, sem1, wsem0, wsem1, asem0, asem1):
        c = lax.axis_index("c")
        t = lax.axis_index("s")
        gidxs = (gidx0, gidx1)
        dstbs = (dstb0, dstb1)
        idxbs = (idxb0, idxb1)
        rowss = (rows0, rows1)
        sems = (sem0, sem1)
        nodebufs = (nodebuf0, nodebuf1)
        wsems = (wsem0, wsem1)
        accbufs = (accbuf0, accbuf1)
        degvs = (degv0, degv1)
        asems = (asem0, asem1)

        def fill_ones(i, carry):
            onesbuf[pl.ds(i * _LN, _LN)] = jnp.ones((_LN,), jnp.float32)
            return carry
        lax.fori_loop(0, _EC // _LN, fill_ones, 0)

        def fill_zvec(i, carry):
            zvec[pl.ds(i * _LN, _LN)] = jnp.zeros((_LN,), jnp.float32)
            return carry
        lax.fori_loop(0, nzd // _LN, fill_zvec, 0)

        for bi in range(bpc):
            b = c * bpc + bi
            bN = b * N

            def zero_rows(i, carry):
                def zr(dc, cr):
                    rows0[i, pl.ds(dc * _LN, _LN)] = jnp.zeros(
                        (_LN,), jnp.float32)
                    return cr
                lax.fori_loop(0, dcols, zr, 0)
                return carry
            lax.fori_loop(0, _EC, zero_rows, 0)
            nfull = nza // _EC
            for j in range(nfull):
                pltpu.sync_copy(
                    rows0, acc_sh.at[pl.ds(t * nza + j * _EC, _EC)])
            rem = nza - nfull * _EC
            if rem:
                pltpu.sync_copy(
                    rows0.at[pl.ds(0, rem)],
                    acc_sh.at[pl.ds(t * nza + nfull * _EC, rem)])
            pltpu.sync_copy(zvec, deg_sh.at[pl.ds(t * nzd, nzd)])
            plsc.subcore_barrier()

            def e_super(s):
                off = (t * cpt_e + s * _SUP) * _EC
                pltpu.sync_copy(src_ref.at[pl.ds(off, _SUP * _EC)], ssrc)
                pltpu.sync_copy(dst_ref.at[pl.ds(off, _SUP * _EC)], sdst)

            def e_ok(i):
                return (i < cpt_e) & (t * cpt_e + i < e_chunks)

            def e_issue(i, p):
                @pl.when(e_ok(i))
                def _():
                    koff = (i % _SUP) * _EC

                    def adj(r, cr):
                        sl = pl.ds(r * _LN, _LN)
                        gidxs[p][sl] = ssrc[pl.ds(koff + r * _LN, _LN)] + bN
                        dstbs[p][sl] = sdst[pl.ds(koff + r * _LN, _LN)]
                        return cr
                    lax.fori_loop(0, _EC // _LN, adj, 0)
                    pltpu.async_copy(x_ref.at[gidxs[p]], rowss[p], sems[p])

            def e_drain(i, p):
                @pl.when(e_ok(i))
                def _():
                    pltpu.make_async_copy(
                        x_ref.at[gidxs[p]], rowss[p], sems[p]).wait()
                    pltpu.sync_copy(rowss[p], acc_sh.at[dstbs[p]], add=True)
                    pltpu.sync_copy(onesbuf, deg_sh.at[dstbs[p]], add=True)

            e_super(0)
            e_issue(0, 0)

            def edge_pair(i2, carry):
                i0 = 2 * i2
                e_issue(i0 + 1, 1)
                e_drain(i0, 0)

                @pl.when((i0 + 2) % _SUP == 0)
                def _():
                    e_super((i0 + 2) // _SUP)
                e_issue(i0 + 2, 0)
                e_drain(i0 + 1, 1)
                return carry
            lax.fori_loop(0, _cdiv(cpt_e, 2), edge_pair, 0)

            plsc.subcore_barrier()

            def a_super(s):
                off = (t * cpt_n + s * _SUP) * _NCH
                pltpu.sync_copy(sel_ref.at[pl.ds(off, _SUP * _NCH)], ssel)

            def a_ok(i):
                return ((i >= 0) & (i < cpt_n)
                        & (t * cpt_n + i < n_chunks))

            def a_issue(i, p):
                @pl.when(a_ok(i))
                def _():
                    roff = (i % _SUP) * _NCH

                    def adj(r, cr):
                        idxbs[p][pl.ds(r * K, K)] = ssel[roff + r, :] + bN
                        return cr
                    lax.fori_loop(0, _NCH, adj, 0)
                    pltpu.async_copy(x_ref.at[idxbs[p]], rowss[p], sems[p])
                    node0 = (t * cpt_n + i) * _NCH
                    pltpu.async_copy(
                        acc_sh.at[pl.ds(node0, _NCH)], accbufs[p], asems[p])
                    pltpu.async_copy(
                        deg_sh.at[pl.ds(node0, _NCH)], degvs[p], asems[p])

            def a_wwait(i, p):
                @pl.when(a_ok(i))
                def _():
                    node0 = (t * cpt_n + i) * _NCH
                    pltpu.make_async_copy(
                        nodebufs[p],
                        out_ref.at[pl.ds(bN + node0, _NCH)],
                        wsems[p]).wait()

            def a_drain(i, p):
                a_wwait(i - 2, p)

                @pl.when(a_ok(i))
                def _():
                    pltpu.make_async_copy(
                        x_ref.at[idxbs[p]], rowss[p], sems[p]).wait()
                    ch = t * cpt_n + i
                    node0 = ch * _NCH
                    pltpu.make_async_copy(
                        acc_sh.at[pl.ds(node0, _NCH)], accbufs[p],
                        asems[p]).wait()
                    pltpu.make_async_copy(
                        deg_sh.at[pl.ds(node0, _NCH)], degvs[p],
                        asems[p]).wait()
                    invbuf[:] = 1.0 / jnp.maximum(degvs[p][:], 1.0)
                    rows = rowss[p]
                    accbuf = accbufs[p]

                    def red(r, cr):
                        base = r * K
                        iv = invbuf[pl.ds(r, 1)][0]
                        for dc in range(dcols):
                            sl = pl.ds(dc * _LN, _LN)
                            s = rows[base, sl]
                            for k in range(1, K):
                                s = s + rows[base + k, sl]
                            nodebufs[p][r, sl] = (s * (1.0 / K)
                                                  + accbuf[r, sl] * iv)
                        return cr
                    lax.fori_loop(0, _NCH, red, 0)
                    pltpu.async_copy(
                        nodebufs[p], out_ref.at[pl.ds(bN + node0, _NCH)],
                        wsems[p])

            a_super(0)
            a_issue(0, 0)

            def node_pair(i2, carry):
                i0 = 2 * i2
                a_issue(i0 + 1, 1)
                a_drain(i0, 0)

                @pl.when((i0 + 2) % _SUP == 0)
                def _():
                    a_super((i0 + 2) // _SUP)
                a_issue(i0 + 2, 0)
                a_drain(i0 + 1, 1)
                return carry
            lax.fori_loop(0, _cdiv(cpt_n, 2), node_pair, 0)
            last = 2 * _cdiv(cpt_n, 2)
            a_wwait(last - 2, 0)
            a_wwait(last - 1, 1)
            if bi + 1 < bpc:
                plsc.subcore_barrier()

    fn = pl.kernel(
        body,
        out_type=jax.ShapeDtypeStruct((B * N, D), jnp.float32),
        mesh=mesh,
        scratch_types=[
            pltpu.VMEM((_SUP * _EC,), jnp.int32),
            pltpu.VMEM((_SUP * _EC,), jnp.int32),
            pltpu.VMEM((_SUP * _NCH, K), jnp.int32),
            pltpu.VMEM((_EC,), jnp.int32),
            pltpu.VMEM((_EC,), jnp.int32),
            pltpu.VMEM((_EC,), jnp.int32),
            pltpu.VMEM((_EC,), jnp.int32),
            pltpu.VMEM((_NCH * K,), jnp.int32),
            pltpu.VMEM((_NCH * K,), jnp.int32),
            pltpu.VMEM((_EC,), jnp.float32),
            pltpu.VMEM((_EC, D), jnp.float32),
            pltpu.VMEM((_EC, D), jnp.float32),
            pltpu.VMEM((_NCH, D), jnp.float32),
            pltpu.VMEM((_NCH, D), jnp.float32),
            pltpu.VMEM((_NCH, D), jnp.float32),
            pltpu.VMEM((_NCH, D), jnp.float32),
            pltpu.VMEM((_NCH,), jnp.float32),
            pltpu.VMEM((_NCH,), jnp.float32),
            pltpu.VMEM((_NCH,), jnp.float32),
            pltpu.VMEM((nzd,), jnp.float32),
            pltpu.VMEM_SHARED((N, D), jnp.float32),
            pltpu.VMEM_SHARED((_NS * nzd,), jnp.float32),
            pltpu.SemaphoreType.DMA,
            pltpu.SemaphoreType.DMA,
            pltpu.SemaphoreType.DMA,
            pltpu.SemaphoreType.DMA,
            pltpu.SemaphoreType.DMA,
            pltpu.SemaphoreType.DMA,
        ],
    )
    return fn(X2, sel_p, src_p, dst_p)


def _tc_body(a_ref, h_ref, wz, bz, lwzt, lwzb, lbz, wr, br, lwrt, lwrb, lbr,
             wh, bh, lwht, lwhb, lbh, out_ref):
    a = a_ref[...]
    h = h_ref[...]
    dot = functools.partial(jnp.dot, preferred_element_type=jnp.float32)
    hz = jnp.maximum(dot(a, wz[...]) + bz[...], 0.0)
    hr = jnp.maximum(dot(a, wr[...]) + br[...], 0.0)
    hh = jnp.maximum(dot(a, wh[...]) + bh[...], 0.0)
    z = jax.nn.sigmoid(dot(hz, lwzt[...]) + dot(h, lwzb[...]) + lbz[...])
    r = jax.nn.sigmoid(dot(hr, lwrt[...]) + dot(h, lwrb[...]) + lbr[...])
    ht = jnp.tanh(dot(hh, lwht[...]) + dot(h * r, lwhb[...]) + lbh[...])
    out_ref[...] = z * h + (1.0 - z) * ht


def _tc_dense(agg2, H2, Wz, bz, LWz, Lbz, Wr, br, LWr, Lbr, Wh, bh, LWh, Lbh):
    M, DO = H2.shape
    DI = agg2.shape[1]
    rblk = 2000
    assert M % rblk == 0
    grid = (M // rblk,)
    row = pl.BlockSpec((rblk, DI), lambda i: (i, 0))
    roww = pl.BlockSpec((rblk, DO), lambda i: (i, 0))
    wfull = lambda s: pl.BlockSpec(s, lambda i: (0, 0))
    b2 = lambda v: v.reshape(1, DO)
    return pl.pallas_call(
        _tc_body,
        grid=grid,
        in_specs=[row, roww] + [wfull((DI, DO)), wfull((1, DO)),
                                wfull((DO, DO)), wfull((DO, DO)), wfull((1, DO))] * 3,
        out_specs=roww,
        out_shape=jax.ShapeDtypeStruct((M, DO), jnp.float32),
    )(agg2, H2,
      Wz, b2(bz), LWz[:DO], LWz[DO:], b2(Lbz),
      Wr, b2(br), LWr[:DO], LWr[DO:], b2(Lbr),
      Wh, b2(bh), LWh[:DO], LWh[DO:], b2(Lbh))


def kernel(X, H, Wz, bz, LWz, Lbz, Wr, br, LWr, Lbr, Wh, bh, LWh, Lbh,
           edge_index, selections):
    B, N, DI = X.shape
    DO = H.shape[-1]
    E = edge_index.shape[1]
    K = selections.shape[1]
    X2 = X.reshape(B * N, DI)
    agg2 = _sc_aggregate(X2, selections, edge_index[0], edge_index[1],
                         B=B, N=N, D=DI, E=E, K=K)
    out2 = _tc_dense(agg2, H.reshape(B * N, DO),
                     Wz, bz, LWz, Lbz, Wr, br, LWr, Lbr, Wh, bh, LWh, Lbh)
    return out2.reshape(B, N, DO)


# speedup vs baseline: 1.0576x; 1.0206x over previous
import functools

import jax
import jax.numpy as jnp
from jax import lax
from jax.experimental import pallas as pl
from jax.experimental.pallas import tpu as pltpu
from jax.experimental.pallas import tpu_sc as plsc

_NC = 2
_NS = 16
_LN = 16
_EC = 128
_NCH = 8
_SUP = 8


def _cdiv(a, b):
    return (a + b - 1) // b


@functools.partial(jax.jit, static_argnames=("B", "N", "D", "E", "K"))
def _sc_aggregate(X2, sel, src, dst, B, N, D, E, K):
    assert B % _NC == 0 and N % _NCH == 0 and E % _EC == 0
    assert K == _LN and D % _LN == 0
    bpc = B // _NC
    n_chunks = N // _NCH
    e_chunks = E // _EC
    cpt_n = _cdiv(n_chunks, _NS)
    cpt_e = _cdiv(e_chunks, _NS)
    dcols = D // _LN
    nza = N // _NS
    nzd = _cdiv(cpt_n * _NCH, _LN) * _LN

    sup_e = _cdiv(cpt_e, _SUP) + 1
    sup_n = _cdiv(cpt_n, _SUP) + 1
    src_p = jnp.pad(src, (0, _NS * sup_e * _SUP * _EC - E))
    dst_p = jnp.pad(dst, (0, _NS * sup_e * _SUP * _EC - E))
    sel_p = jnp.pad(sel, ((0, _NS * sup_n * _SUP * _NCH - N), (0, 0)))

    mesh = plsc.VectorSubcoreMesh(core_axis_name="c", subcore_axis_name="s")

    def body(x_ref, sel_ref, src_ref, dst_ref, out_ref,
             ssrc, sdst, ssel, gidx0, gidx1, dstb0, dstb1, idxb0, idxb1,
             onesbuf, rows0, rows1, nodebuf0, nodebuf1, accbuf0, accbuf1,
             degv0, degv1, invbuf, zvec, acc_sh, deg_sh,
             sem0, sem1, wsem0, wsem1, asem0, asem1, esem0, esem1):
        c = lax.axis_index("c")
        t = lax.axis_index("s")
        gidxs = (gidx0, gidx1)
        dstbs = (dstb0, dstb1)
        idxbs = (idxb0, idxb1)
        rowss = (rows0, rows1)
        sems = (sem0, sem1)
        esems = (esem0, esem1)
        nodebufs = (nodebuf0, nodebuf1)
        wsems = (wsem0, wsem1)
        accbufs = (accbuf0, accbuf1)
        degvs = (degv0, degv1)
        asems = (asem0, asem1)

        def fill_ones(i, carry):
            onesbuf[pl.ds(i * _LN, _LN)] = jnp.ones((_LN,), jnp.float32)
            return carry
        lax.fori_loop(0, _EC // _LN, fill_ones, 0)

        def fill_zvec(i, carry):
            zvec[pl.ds(i * _LN, _LN)] = jnp.zeros((_LN,), jnp.float32)
            return carry
        lax.fori_loop(0, nzd // _LN, fill_zvec, 0)

        for bi in range(bpc):
            b = c * bpc + bi
            bN = b * N

            def zero_rows(i, carry):
                def zr(dc, cr):
                    rows0[i, pl.ds(dc * _LN, _LN)] = jnp.zeros(
                        (_LN,), jnp.float32)
                    return cr
                lax.fori_loop(0, dcols, zr, 0)
                return carry
            lax.fori_loop(0, _EC, zero_rows, 0)
            nfull = nza // _EC
            for j in range(nfull):
                pltpu.sync_copy(
                    rows0, acc_sh.at[pl.ds(t * nza + j * _EC, _EC)])
            rem = nza - nfull * _EC
            if rem:
                pltpu.sync_copy(
                    rows0.at[pl.ds(0, rem)],
                    acc_sh.at[pl.ds(t * nza + nfull * _EC, rem)])
            pltpu.sync_copy(zvec, deg_sh.at[pl.ds(t * nzd, nzd)])
            plsc.subcore_barrier()

            def e_super(s):
                off = (t * cpt_e + s * _SUP) * _EC
                pltpu.sync_copy(src_ref.at[pl.ds(off, _SUP * _EC)], ssrc)
                pltpu.sync_copy(dst_ref.at[pl.ds(off, _SUP * _EC)], sdst)

            def e_ok(i):
                return ((i >= 0) & (i < cpt_e)
                        & (t * cpt_e + i < e_chunks))

            def e_swait(i, p):
                @pl.when(e_ok(i))
                def _():
                    pltpu.make_async_copy(
                        rowss[p], acc_sh.at[dstbs[p]], esems[p]).wait()

            def e_issue(i, p):
                e_swait(i - 2, p)

                @pl.when(e_ok(i))
                def _():
                    koff = (i % _SUP) * _EC

                    def adj(r, cr):
                        sl = pl.ds(r * _LN, _LN)
                        gidxs[p][sl] = ssrc[pl.ds(koff + r * _LN, _LN)] + bN
                        dstbs[p][sl] = sdst[pl.ds(koff + r * _LN, _LN)]
                        return cr
                    lax.fori_loop(0, _EC // _LN, adj, 0)
                    pltpu.async_copy(x_ref.at[gidxs[p]], rowss[p], sems[p])

            def e_drain(i, p):
                @pl.when(e_ok(i))
                def _():
                    pltpu.make_async_copy(
                        x_ref.at[gidxs[p]], rowss[p], sems[p]).wait()
                    pltpu.async_copy(rowss[p], acc_sh.at[dstbs[p]],
                                     esems[p], add=True)
                    pltpu.sync_copy(onesbuf, deg_sh.at[dstbs[p]], add=True)

            e_super(0)
            e_issue(0, 0)

            def edge_pair(i2, carry):
                i0 = 2 * i2
                e_issue(i0 + 1, 1)
                e_drain(i0, 0)

                @pl.when((i0 + 2) % _SUP == 0)
                def _():
                    e_super((i0 + 2) // _SUP)
                e_issue(i0 + 2, 0)
                e_drain(i0 + 1, 1)
                return carry
            lax.fori_loop(0, _cdiv(cpt_e, 2), edge_pair, 0)
            e_swait(2 * _cdiv(cpt_e, 2) - 1, 1)

            plsc.subcore_barrier()

            def a_super(s):
                off = (t * cpt_n + s * _SUP) * _NCH
                pltpu.sync_copy(sel_ref.at[pl.ds(off, _SUP * _NCH)], ssel)

            def a_ok(i):
                return ((i >= 0) & (i < cpt_n)
                        & (t * cpt_n + i < n_chunks))

            def a_issue(i, p):
                @pl.when(a_ok(i))
                def _():
                    roff = (i % _SUP) * _NCH

                    def adj(r, cr):
                        idxbs[p][pl.ds(r * K, K)] = ssel[roff + r, :] + bN
                        return cr
                    lax.fori_loop(0, _NCH, adj, 0)
                    pltpu.async_copy(x_ref.at[idxbs[p]], rowss[p], sems[p])
                    node0 = (t * cpt_n + i) * _NCH
                    pltpu.async_copy(
                        acc_sh.at[pl.ds(node0, _NCH)], accbufs[p], asems[p])
                    pltpu.async_copy(
                        deg_sh.at[pl.ds(node0, _NCH)], degvs[p], asems[p])

            def a_wwait(i, p):
                @pl.when(a_ok(i))
                def _():
                    node0 = (t * cpt_n + i) * _NCH
                    pltpu.make_async_copy(
                        nodebufs[p],
                        out_ref.at[pl.ds(bN + node0, _NCH)],
                        wsems[p]).wait()

            def a_drain(i, p):
                a_wwait(i - 2, p)

                @pl.when(a_ok(i))
                def _():
                    pltpu.make_async_copy(
                        x_ref.at[idxbs[p]], rowss[p], sems[p]).wait()
                    ch = t * cpt_n + i
                    node0 = ch * _NCH
                    pltpu.make_async_copy(
                        acc_sh.at[pl.ds(node0, _NCH)], accbufs[p],
                        asems[p]).wait()
                    pltpu.make_async_copy(
                        deg_sh.at[pl.ds(node0, _NCH)], degvs[p],
                        asems[p]).wait()
                    invbuf[:] = 1.0 / jnp.maximum(degvs[p][:], 1.0)
                    rows = rowss[p]
                    accbuf = accbufs[p]

                    def red(r, cr):
                        base = r * K
                        iv = invbuf[pl.ds(r, 1)][0]
                        for dc in range(dcols):
                            sl = pl.ds(dc * _LN, _LN)
                            s = rows[base, sl]
                            for k in range(1, K):
                                s = s + rows[base + k, sl]
                            nodebufs[p][r, sl] = (s * (1.0 / K)
                                                  + accbuf[r, sl] * iv)
                        return cr
                    lax.fori_loop(0, _NCH, red, 0)
                    pltpu.async_copy(
                        nodebufs[p], out_ref.at[pl.ds(bN + node0, _NCH)],
                        wsems[p])

            a_super(0)
            a_issue(0, 0)

            def node_pair(i2, carry):
                i0 = 2 * i2
                a_issue(i0 + 1, 1)
                a_drain(i0, 0)

                @pl.when((i0 + 2) % _SUP == 0)
                def _():
                    a_super((i0 + 2) // _SUP)
                a_issue(i0 + 2, 0)
                a_drain(i0 + 1, 1)
                return carry
            lax.fori_loop(0, _cdiv(cpt_n, 2), node_pair, 0)
            last = 2 * _cdiv(cpt_n, 2)
            a_wwait(last - 2, 0)
            a_wwait(last - 1, 1)
            if bi + 1 < bpc:
                plsc.subcore_barrier()

    fn = pl.kernel(
        body,
        out_type=jax.ShapeDtypeStruct((B * N, D), jnp.float32),
        mesh=mesh,
        scratch_types=[
            pltpu.VMEM((_SUP * _EC,), jnp.int32),
            pltpu.VMEM((_SUP * _EC,), jnp.int32),
            pltpu.VMEM((_SUP * _NCH, K), jnp.int32),
            pltpu.VMEM((_EC,), jnp.int32),
            pltpu.VMEM((_EC,), jnp.int32),
            pltpu.VMEM((_EC,), jnp.int32),
            pltpu.VMEM((_EC,), jnp.int32),
            pltpu.VMEM((_NCH * K,), jnp.int32),
            pltpu.VMEM((_NCH * K,), jnp.int32),
            pltpu.VMEM((_EC,), jnp.float32),
            pltpu.VMEM((_EC, D), jnp.float32),
            pltpu.VMEM((_EC, D), jnp.float32),
            pltpu.VMEM((_NCH, D), jnp.float32),
            pltpu.VMEM((_NCH, D), jnp.float32),
            pltpu.VMEM((_NCH, D), jnp.float32),
            pltpu.VMEM((_NCH, D), jnp.float32),
            pltpu.VMEM((_NCH,), jnp.float32),
            pltpu.VMEM((_NCH,), jnp.float32),
            pltpu.VMEM((_NCH,), jnp.float32),
            pltpu.VMEM((nzd,), jnp.float32),
            pltpu.VMEM_SHARED((N, D), jnp.float32),
            pltpu.VMEM_SHARED((_NS * nzd,), jnp.float32),
            pltpu.SemaphoreType.DMA,
            pltpu.SemaphoreType.DMA,
            pltpu.SemaphoreType.DMA,
            pltpu.SemaphoreType.DMA,
            pltpu.SemaphoreType.DMA,
            pltpu.SemaphoreType.DMA,
            pltpu.SemaphoreType.DMA,
            pltpu.SemaphoreType.DMA,
        ],
    )
    return fn(X2, sel_p, src_p, dst_p)


def _tc_body(a_ref, h_ref, wz, bz, lwzt, lwzb, lbz, wr, br, lwrt, lwrb, lbr,
             wh, bh, lwht, lwhb, lbh, out_ref):
    a = a_ref[...]
    h = h_ref[...]
    dot = functools.partial(jnp.dot, preferred_element_type=jnp.float32)
    hz = jnp.maximum(dot(a, wz[...]) + bz[...], 0.0)
    hr = jnp.maximum(dot(a, wr[...]) + br[...], 0.0)
    hh = jnp.maximum(dot(a, wh[...]) + bh[...], 0.0)
    z = jax.nn.sigmoid(dot(hz, lwzt[...]) + dot(h, lwzb[...]) + lbz[...])
    r = jax.nn.sigmoid(dot(hr, lwrt[...]) + dot(h, lwrb[...]) + lbr[...])
    ht = jnp.tanh(dot(hh, lwht[...]) + dot(h * r, lwhb[...]) + lbh[...])
    out_ref[...] = z * h + (1.0 - z) * ht


def _tc_dense(agg2, H2, Wz, bz, LWz, Lbz, Wr, br, LWr, Lbr, Wh, bh, LWh, Lbh):
    M, DO = H2.shape
    DI = agg2.shape[1]
    rblk = 2000
    assert M % rblk == 0
    grid = (M // rblk,)
    row = pl.BlockSpec((rblk, DI), lambda i: (i, 0))
    roww = pl.BlockSpec((rblk, DO), lambda i: (i, 0))
    wfull = lambda s: pl.BlockSpec(s, lambda i: (0, 0))
    b2 = lambda v: v.reshape(1, DO)
    return pl.pallas_call(
        _tc_body,
        grid=grid,
        in_specs=[row, roww] + [wfull((DI, DO)), wfull((1, DO)),
                                wfull((DO, DO)), wfull((DO, DO)), wfull((1, DO))] * 3,
        out_specs=roww,
        out_shape=jax.ShapeDtypeStruct((M, DO), jnp.float32),
    )(agg2, H2,
      Wz, b2(bz), LWz[:DO], LWz[DO:], b2(Lbz),
      Wr, b2(br), LWr[:DO], LWr[DO:], b2(Lbr),
      Wh, b2(bh), LWh[:DO], LWh[DO:], b2(Lbh))


def kernel(X, H, Wz, bz, LWz, Lbz, Wr, br, LWr, Lbr, Wh, bh, LWh, Lbh,
           edge_index, selections):
    B, N, DI = X.shape
    DO = H.shape[-1]
    E = edge_index.shape[1]
    K = selections.shape[1]
    X2 = X.reshape(B * N, DI)
    agg2 = _sc_aggregate(X2, selections, edge_index[0], edge_index[1],
                         B=B, N=N, D=DI, E=E, K=K)
    out2 = _tc_dense(agg2, H.reshape(B * N, DO),
                     Wz, bz, LWz, Lbz, Wr, br, LWr, Lbr, Wh, bh, LWh, Lbh)
    return out2.reshape(B, N, DO)


# speedup vs baseline: 1.0601x; 1.0023x over previous
import functools

import jax
import jax.numpy as jnp
from jax import lax
from jax.experimental import pallas as pl
from jax.experimental.pallas import tpu as pltpu
from jax.experimental.pallas import tpu_sc as plsc

_NC = 2
_NS = 16
_LN = 16
_EC = 128
_NCH = 8
_SUP = 8


def _cdiv(a, b):
    return (a + b - 1) // b


@functools.partial(jax.jit, static_argnames=("B", "N", "D", "E", "K"))
def _sc_aggregate(X2, sel, src, dst, B, N, D, E, K):
    assert B % _NC == 0 and N % _NCH == 0 and E % _EC == 0
    assert K == _LN and D % _LN == 0
    bpc = B // _NC
    n_chunks = N // _NCH
    e_chunks = E // _EC
    cpt_n = _cdiv(n_chunks, _NS)
    cpt_e = _cdiv(e_chunks, _NS)
    dcols = D // _LN
    nza = N // _NS
    nzd = _cdiv(cpt_n * _NCH, _LN) * _LN

    sup_e = _cdiv(cpt_e, _SUP) + 1
    sup_n = _cdiv(cpt_n, _SUP) + 1
    src_p = jnp.pad(src, (0, _NS * sup_e * _SUP * _EC - E))
    dst_p = jnp.pad(dst, (0, _NS * sup_e * _SUP * _EC - E))
    sel_p = jnp.pad(sel, ((0, _NS * sup_n * _SUP * _NCH - N), (0, 0)))

    mesh = plsc.VectorSubcoreMesh(core_axis_name="c", subcore_axis_name="s")

    def body(x_ref, sel_ref, src_ref, dst_ref, out_ref,
             ssrc, sdst, ssel, gidx0, gidx1, dstb0, dstb1, idxb0, idxb1,
             onesbuf, rows0, rows1, nodebuf0, nodebuf1, accbuf0, accbuf1,
             degv0, degv1, invbuf, zvec, acc_sh, deg_sh,
             sem0, sem1, wsem0, wsem1, asem0, asem1, esem0, esem1,
             dsem0, dsem1):
        c = lax.axis_index("c")
        t = lax.axis_index("s")
        gidxs = (gidx0, gidx1)
        dstbs = (dstb0, dstb1)
        idxbs = (idxb0, idxb1)
        rowss = (rows0, rows1)
        sems = (sem0, sem1)
        esems = (esem0, esem1)
        dsems = (dsem0, dsem1)
        nodebufs = (nodebuf0, nodebuf1)
        wsems = (wsem0, wsem1)
        accbufs = (accbuf0, accbuf1)
        degvs = (degv0, degv1)
        asems = (asem0, asem1)

        def fill_ones(i, carry):
            onesbuf[pl.ds(i * _LN, _LN)] = jnp.ones((_LN,), jnp.float32)
            return carry
        lax.fori_loop(0, _EC // _LN, fill_ones, 0)

        def fill_zvec(i, carry):
            zvec[pl.ds(i * _LN, _LN)] = jnp.zeros((_LN,), jnp.float32)
            return carry
        lax.fori_loop(0, nzd // _LN, fill_zvec, 0)

        for bi in range(bpc):
            b = c * bpc + bi
            bN = b * N

            def zero_rows(i, carry):
                def zr(dc, cr):
                    rows0[i, pl.ds(dc * _LN, _LN)] = jnp.zeros(
                        (_LN,), jnp.float32)
                    return cr
                lax.fori_loop(0, dcols, zr, 0)
                return carry
            lax.fori_loop(0, _EC, zero_rows, 0)
            nfull = nza // _EC
            for j in range(nfull):
                pltpu.sync_copy(
                    rows0, acc_sh.at[pl.ds(t * nza + j * _EC, _EC)])
            rem = nza - nfull * _EC
            if rem:
                pltpu.sync_copy(
                    rows0.at[pl.ds(0, rem)],
                    acc_sh.at[pl.ds(t * nza + nfull * _EC, rem)])
            pltpu.sync_copy(zvec, deg_sh.at[pl.ds(t * nzd, nzd)])
            plsc.subcore_barrier()

            def e_super(s):
                off = (t * cpt_e + s * _SUP) * _EC
                pltpu.sync_copy(src_ref.at[pl.ds(off, _SUP * _EC)], ssrc)
                pltpu.sync_copy(dst_ref.at[pl.ds(off, _SUP * _EC)], sdst)

            def e_ok(i):
                return ((i >= 0) & (i < cpt_e)
                        & (t * cpt_e + i < e_chunks))

            def e_swait(i, p):
                @pl.when(e_ok(i))
                def _():
                    pltpu.make_async_copy(
                        rowss[p], acc_sh.at[dstbs[p]], esems[p]).wait()
                    pltpu.make_async_copy(
                        onesbuf, deg_sh.at[dstbs[p]], dsems[p]).wait()

            def e_issue(i, p):
                e_swait(i - 2, p)

                @pl.when(e_ok(i))
                def _():
                    koff = (i % _SUP) * _EC

                    def adj(r, cr):
                        sl = pl.ds(r * _LN, _LN)
                        gidxs[p][sl] = ssrc[pl.ds(koff + r * _LN, _LN)] + bN
                        dstbs[p][sl] = sdst[pl.ds(koff + r * _LN, _LN)]
                        return cr
                    lax.fori_loop(0, _EC // _LN, adj, 0)
                    pltpu.async_copy(x_ref.at[gidxs[p]], rowss[p], sems[p])

            def e_drain(i, p):
                @pl.when(e_ok(i))
                def _():
                    pltpu.make_async_copy(
                        x_ref.at[gidxs[p]], rowss[p], sems[p]).wait()
                    pltpu.async_copy(rowss[p], acc_sh.at[dstbs[p]],
                                     esems[p], add=True)
                    pltpu.async_copy(onesbuf, deg_sh.at[dstbs[p]],
                                     dsems[p], add=True)

            e_super(0)
            e_issue(0, 0)

            def edge_pair(i2, carry):
                i0 = 2 * i2
                e_issue(i0 + 1, 1)
                e_drain(i0, 0)

                @pl.when((i0 + 2) % _SUP == 0)
                def _():
                    e_super((i0 + 2) // _SUP)
                e_issue(i0 + 2, 0)
                e_drain(i0 + 1, 1)
                return carry
            lax.fori_loop(0, _cdiv(cpt_e, 2), edge_pair, 0)
            e_swait(2 * _cdiv(cpt_e, 2) - 1, 1)

            plsc.subcore_barrier()

            def a_super(s):
                off = (t * cpt_n + s * _SUP) * _NCH
                pltpu.sync_copy(sel_ref.at[pl.ds(off, _SUP * _NCH)], ssel)

            def a_ok(i):
                return ((i >= 0) & (i < cpt_n)
                        & (t * cpt_n + i < n_chunks))

            def a_issue(i, p):
                @pl.when(a_ok(i))
                def _():
                    roff = (i % _SUP) * _NCH

                    def adj(r, cr):
                        idxbs[p][pl.ds(r * K, K)] = ssel[roff + r, :] + bN
                        return cr
                    lax.fori_loop(0, _NCH, adj, 0)
                    pltpu.async_copy(x_ref.at[idxbs[p]], rowss[p], sems[p])
                    node0 = (t * cpt_n + i) * _NCH
                    pltpu.async_copy(
                        acc_sh.at[pl.ds(node0, _NCH)], accbufs[p], asems[p])
                    pltpu.async_copy(
                        deg_sh.at[pl.ds(node0, _NCH)], degvs[p], asems[p])

            def a_wwait(i, p):
                @pl.when(a_ok(i))
                def _():
                    node0 = (t * cpt_n + i) * _NCH
                    pltpu.make_async_copy(
                        nodebufs[p],
                        out_ref.at[pl.ds(bN + node0, _NCH)],
                        wsems[p]).wait()

            def a_drain(i, p):
                a_wwait(i - 2, p)

                @pl.when(a_ok(i))
                def _():
                    pltpu.make_async_copy(
                        x_ref.at[idxbs[p]], rowss[p], sems[p]).wait()
                    ch = t * cpt_n + i
                    node0 = ch * _NCH
                    pltpu.make_async_copy(
                        acc_sh.at[pl.ds(node0, _NCH)], accbufs[p],
                        asems[p]).wait()
                    pltpu.make_async_copy(
                        deg_sh.at[pl.ds(node0, _NCH)], degvs[p],
                        asems[p]).wait()
                    invbuf[:] = 1.0 / jnp.maximum(degvs[p][:], 1.0)
                    rows = rowss[p]
                    accbuf = accbufs[p]

                    def red(r, cr):
                        base = r * K
                        iv = invbuf[pl.ds(r, 1)][0]
                        for dc in range(dcols):
                            sl = pl.ds(dc * _LN, _LN)
                            s = rows[base, sl]
                            for k in range(1, K):
                                s = s + rows[base + k, sl]
                            nodebufs[p][r, sl] = (s * (1.0 / K)
                                                  + accbuf[r, sl] * iv)
                        return cr
                    lax.fori_loop(0, _NCH, red, 0)
                    pltpu.async_copy(
                        nodebufs[p], out_ref.at[pl.ds(bN + node0, _NCH)],
                        wsems[p])

            a_super(0)
            a_issue(0, 0)

            def node_pair(i2, carry):
                i0 = 2 * i2
                a_issue(i0 + 1, 1)
                a_drain(i0, 0)

                @pl.when((i0 + 2) % _SUP == 0)
                def _():
                    a_super((i0 + 2) // _SUP)
                a_issue(i0 + 2, 0)
                a_drain(i0 + 1, 1)
                return carry
            lax.fori_loop(0, _cdiv(cpt_n, 2), node_pair, 0)
            last = 2 * _cdiv(cpt_n, 2)
            a_wwait(last - 2, 0)
            a_wwait(last - 1, 1)
            if bi + 1 < bpc:
                plsc.subcore_barrier()

    fn = pl.kernel(
        body,
        out_type=jax.ShapeDtypeStruct((B * N, D), jnp.float32),
        mesh=mesh,
        scratch_types=[
            pltpu.VMEM((_SUP * _EC,), jnp.int32),
            pltpu.VMEM((_SUP * _EC,), jnp.int32),
            pltpu.VMEM((_SUP * _NCH, K), jnp.int32),
            pltpu.VMEM((_EC,), jnp.int32),
            pltpu.VMEM((_EC,), jnp.int32),
            pltpu.VMEM((_EC,), jnp.int32),
            pltpu.VMEM((_EC,), jnp.int32),
            pltpu.VMEM((_NCH * K,), jnp.int32),
            pltpu.VMEM((_NCH * K,), jnp.int32),
            pltpu.VMEM((_EC,), jnp.float32),
            pltpu.VMEM((_EC, D), jnp.float32),
            pltpu.VMEM((_EC, D), jnp.float32),
            pltpu.VMEM((_NCH, D), jnp.float32),
            pltpu.VMEM((_NCH, D), jnp.float32),
            pltpu.VMEM((_NCH, D), jnp.float32),
            pltpu.VMEM((_NCH, D), jnp.float32),
            pltpu.VMEM((_NCH,), jnp.float32),
            pltpu.VMEM((_NCH,), jnp.float32),
            pltpu.VMEM((_NCH,), jnp.float32),
            pltpu.VMEM((nzd,), jnp.float32),
            pltpu.VMEM_SHARED((N, D), jnp.float32),
            pltpu.VMEM_SHARED((_NS * nzd,), jnp.float32),
            pltpu.SemaphoreType.DMA,
            pltpu.SemaphoreType.DMA,
            pltpu.SemaphoreType.DMA,
            pltpu.SemaphoreType.DMA,
            pltpu.SemaphoreType.DMA,
            pltpu.SemaphoreType.DMA,
            pltpu.SemaphoreType.DMA,
            pltpu.SemaphoreType.DMA,
            pltpu.SemaphoreType.DMA,
            pltpu.SemaphoreType.DMA,
        ],
    )
    return fn(X2, sel_p, src_p, dst_p)


def _tc_body(a_ref, h_ref, wz, bz, lwzt, lwzb, lbz, wr, br, lwrt, lwrb, lbr,
             wh, bh, lwht, lwhb, lbh, out_ref):
    a = a_ref[...]
    h = h_ref[...]
    dot = functools.partial(jnp.dot, preferred_element_type=jnp.float32)
    hz = jnp.maximum(dot(a, wz[...]) + bz[...], 0.0)
    hr = jnp.maximum(dot(a, wr[...]) + br[...], 0.0)
    hh = jnp.maximum(dot(a, wh[...]) + bh[...], 0.0)
    z = jax.nn.sigmoid(dot(hz, lwzt[...]) + dot(h, lwzb[...]) + lbz[...])
    r = jax.nn.sigmoid(dot(hr, lwrt[...]) + dot(h, lwrb[...]) + lbr[...])
    ht = jnp.tanh(dot(hh, lwht[...]) + dot(h * r, lwhb[...]) + lbh[...])
    out_ref[...] = z * h + (1.0 - z) * ht


def _tc_dense(agg2, H2, Wz, bz, LWz, Lbz, Wr, br, LWr, Lbr, Wh, bh, LWh, Lbh):
    M, DO = H2.shape
    DI = agg2.shape[1]
    rblk = 2000
    assert M % rblk == 0
    grid = (M // rblk,)
    row = pl.BlockSpec((rblk, DI), lambda i: (i, 0))
    roww = pl.BlockSpec((rblk, DO), lambda i: (i, 0))
    wfull = lambda s: pl.BlockSpec(s, lambda i: (0, 0))
    b2 = lambda v: v.reshape(1, DO)
    return pl.pallas_call(
        _tc_body,
        grid=grid,
        in_specs=[row, roww] + [wfull((DI, DO)), wfull((1, DO)),
                                wfull((DO, DO)), wfull((DO, DO)), wfull((1, DO))] * 3,
        out_specs=roww,
        out_shape=jax.ShapeDtypeStruct((M, DO), jnp.float32),
    )(agg2, H2,
      Wz, b2(bz), LWz[:DO], LWz[DO:], b2(Lbz),
      Wr, b2(br), LWr[:DO], LWr[DO:], b2(Lbr),
      Wh, b2(bh), LWh[:DO], LWh[DO:], b2(Lbh))


def kernel(X, H, Wz, bz, LWz, Lbz, Wr, br, LWr, Lbr, Wh, bh, LWh, Lbh,
           edge_index, selections):
    B, N, DI = X.shape
    DO = H.shape[-1]
    E = edge_index.shape[1]
    K = selections.shape[1]
    X2 = X.reshape(B * N, DI)
    agg2 = _sc_aggregate(X2, selections, edge_index[0], edge_index[1],
                         B=B, N=N, D=DI, E=E, K=K)
    out2 = _tc_dense(agg2, H.reshape(B * N, DO),
                     Wz, bz, LWz, Lbz, Wr, br, LWr, Lbr, Wh, bh, LWh, Lbh)
    return out2.reshape(B, N, DO)
